# Initial kernel scaffold; baseline (speedup 1.0000x reference)
import jax
import jax.numpy as jnp
from jax.experimental import pallas as pl


def kernel(x, gate, lin1_weight, lin2_weight):
    raise NotImplementedError("write your pallas kernel here")


# speedup vs baseline: 2.7374x; 2.7374x over previous
import functools

import jax
import jax.numpy as jnp
from jax import lax
from jax.experimental import pallas as pl
from jax.experimental.pallas import tpu as pltpu

TOPK = 256


def _select_body(x_ref, gate_ref, w_ref, logits_scr):
    i = pl.program_id(0)
    nt = pl.num_programs(0)
    tb = x_ref.shape[0]
    xb = x_ref[...]
    rs = jnp.sum(xb, axis=1, keepdims=True)
    logits_scr[pl.ds(i * tb, tb), :] = rs * gate_ref[...]

    @pl.when(i == nt - 1)
    def _():
        L = logits_scr[...]
        T, E = L.shape
        m = jnp.max(L, axis=0, keepdims=True)
        p = jnp.exp(L - m)
        denom = jnp.sum(p, axis=0, keepdims=True)
        v = p / denom
        key = lax.bitcast_convert_type(v, jnp.int32)

        def bit_step(b, thr):
            cand = thr | (jnp.int32(1) << (30 - b))
            cnt = jnp.sum((key >= cand).astype(jnp.int32), axis=0,
                          keepdims=True)
            return jnp.where(cnt >= TOPK, cand, thr)

        thr = lax.fori_loop(0, 31, bit_step, jnp.zeros((1, E), jnp.int32))

        n_gt = jnp.sum((key > thr).astype(jnp.int32), axis=0, keepdims=True)
        r = TOPK - n_gt
        eq = key == thr
        idx = lax.broadcasted_iota(jnp.int32, (T, E), 0)

        def idx_step(_, lo_hi):
            lo, hi = lo_hi
            mid = (lo + hi) // 2
            cnt = jnp.sum((eq & (idx <= mid)).astype(jnp.int32), axis=0,
                          keepdims=True)
            take = cnt >= r
            return jnp.where(take, lo, mid + 1), jnp.where(take, mid, hi)

        lo = jnp.zeros((1, E), jnp.int32)
        hi = jnp.full((1, E), T - 1, jnp.int32)
        lo, _ = lax.fori_loop(0, 13, idx_step, (lo, hi))

        sel = (key > thr) | (eq & (idx <= lo))
        w_ref[...] = jnp.where(sel, v, 0.0)


def _ffn_body(x_ref, w1_ref, w2_ref, wgt_ref, out_ref, *, eb):
    j = pl.program_id(1)
    n_experts = wgt_ref.shape[1]
    es = w1_ref.shape[1] // eb
    h = jnp.dot(x_ref[...], w1_ref[...],
                preferred_element_type=jnp.float32)
    h = jnp.maximum(h, 0.0)
    rows = lax.broadcasted_iota(jnp.int32, (n_experts, eb * es), 0)
    cols = lax.broadcasted_iota(jnp.int32, (n_experts, eb * es), 1)
    sexp = (cols // es + j * eb == rows).astype(jnp.float32)
    wexp = jnp.dot(wgt_ref[...], sexp, preferred_element_type=jnp.float32)
    z = jnp.dot(h * wexp, w2_ref[...], preferred_element_type=jnp.float32)

    @pl.when(j == 0)
    def _():
        out_ref[...] = z

    @pl.when(j > 0)
    def _():
        out_ref[...] += z


def kernel(x, gate, lin1_weight, lin2_weight):
    batch, cutoff, d = x.shape
    n_experts = gate.shape[1]
    es = lin1_weight.shape[2]
    t_total = batch * cutoff
    x_flat = x.reshape(t_total, d)

    tb = cutoff
    w = pl.pallas_call(
        _select_body,
        grid=(batch,),
        in_specs=[
            pl.BlockSpec((tb, d), lambda i: (i, 0)),
            pl.BlockSpec((cutoff, n_experts), lambda i: (0, 0)),
        ],
        out_specs=pl.BlockSpec((t_total, n_experts), lambda i: (0, 0)),
        out_shape=jax.ShapeDtypeStruct((t_total, n_experts), jnp.float32),
        scratch_shapes=[pltpu.VMEM((t_total, n_experts), jnp.float32)],
    )(x_flat, gate)

    eb = 8
    tb2 = 2048
    w1 = lin1_weight.reshape(d, n_experts * es)
    w2 = lin2_weight.reshape(n_experts * es, d)
    z = pl.pallas_call(
        functools.partial(_ffn_body, eb=eb),
        grid=(t_total // tb2, n_experts // eb),
        in_specs=[
            pl.BlockSpec((tb2, d), lambda i, j: (i, 0)),
            pl.BlockSpec((d, eb * es), lambda i, j: (0, j)),
            pl.BlockSpec((eb * es, d), lambda i, j: (j, 0)),
            pl.BlockSpec((tb2, n_experts), lambda i, j: (i, 0)),
        ],
        out_specs=pl.BlockSpec((tb2, d), lambda i, j: (i, 0)),
        out_shape=jax.ShapeDtypeStruct((t_total, d), jnp.float32),
    )(x_flat, w1, w2, w)

    return z.reshape(batch, cutoff, d)
